# baseline (speedup 1.0000x reference)
import functools

import jax
import jax.numpy as jnp
from jax import lax
from jax.experimental import pallas as pl
from jax.experimental.pallas import tpu as pltpu
from jax.experimental.pallas import tpu_sc as plsc

BATCH = 16384
EMBED = 64
HIDDEN = 128
LANES = 128
NCOLS = 7813
NC = 2
NS = 16
NW = NC * NS
LO_STEP = 244
RANGE = 256
NPAIR = RANGE // 2
NSUB = 8
SUBW = RANGE // NSUB
CAPF = 768
NL = CAPF // 16
SUBCAP = 192
NSL = SUBCAP // 16
STAGE = 128
FLUSH_AT = 96
OUT_ROWS = BATCH + STAGE
DEPTH = 4
BM = 2048


def _sc_body(uid_hbm, iid_hbm, ut_hbm, it_hbm, ou_hbm, oi_hbm,
             idsall, ml_id, ml_pos, scol, slane, spos,
             buf0, buf1, buf2, buf3, rows_st, pos_st, mst_lane, mst_pos,
             sem0, sem1, sem2, sem3):
    wid = lax.axis_index("s") * NC + lax.axis_index("c")
    lo = wid * LO_STEP
    bufs = (buf0, buf1, buf2, buf3)
    sems = (sem0, sem1, sem2, sem3)
    iota16 = lax.iota(jnp.int32, 16)
    zeros16 = jnp.zeros((16,), jnp.int32)
    padv = jnp.full((16,), BATCH, jnp.int32)

    for q in range(48 // 16):
        mst_lane[pl.ds(16 * q, 16)] = zeros16

    def fire(tbl, idx, slot):
        c0 = jnp.minimum(lo + 2 * idx, NCOLS - 2)
        off = pl.multiple_of(c0 * LANES, LANES)
        pltpu.async_copy(tbl.at[:, pl.ds(off, 2 * LANES)], bufs[slot], sems[slot])

    def drain(slot):
        pltpu.make_async_copy(ut_hbm.at[:, pl.ds(0, 2 * LANES)],
                              bufs[slot], sems[slot]).wait()

    def build_lists():
        neg1 = jnp.full((16,), -1, jnp.int32)
        for s in range(NSUB):
            for v in range(NSL):
                scol[s, pl.ds(16 * v, 16)] = neg1

        def scan(i, cnt):
            w = pl.ds(pl.multiple_of(i * 16, 16), 16)
            idv = idsall[w]
            cv = lax.shift_right_logical(idv, 7)
            m = (cv >= lo) & (cv < lo + RANGE)
            plsc.store_compressed(ml_id.at[pl.ds(cnt, 16)], idv, mask=m)
            plsc.store_compressed(ml_pos.at[pl.ds(cnt, 16)], iota16 + i * 16, mask=m)
            pc = plsc.all_reduce_population_count(m)[0]
            return jnp.minimum(cnt + pc, CAPF - 16)

        cnt = lax.fori_loop(0, BATCH // 16, scan, 0, unroll=False)

        for s in range(NSUB):
            def split(v, scnt, s=s):
                w = pl.ds(pl.multiple_of(v * 16, 16), 16)
                idv = ml_id[w]
                posv = ml_pos[w]
                cv = lax.shift_right_logical(idv, 7)
                valid = (iota16 + v * 16) < cnt
                m = valid & (cv >= lo + SUBW * s) & (cv < lo + SUBW * (s + 1))
                plsc.store_compressed(scol.at[s, pl.ds(scnt, 16)], cv, mask=m)
                plsc.store_compressed(slane.at[s, pl.ds(scnt, 16)],
                                      jnp.bitwise_and(idv, LANES - 1), mask=m)
                plsc.store_compressed(spos.at[s, pl.ds(scnt, 16)], posv, mask=m)
                pc = plsc.all_reduce_population_count(m)[0]
                return jnp.minimum(scnt + pc, SUBCAP - 16)

            lax.fori_loop(0, NL, split, 0, unroll=False)

    def process_table(tbl, out_hbm):
        for q in range(STAGE // 16):
            pos_st[0, pl.ds(16 * q, 16)] = padv

        def flush():
            pltpu.sync_copy(rows_st, out_hbm.at[pos_st.at[0]])
            for q in range(STAGE // 16):
                pos_st[0, pl.ds(16 * q, 16)] = padv

        def do_pair(idx, k, outcnt):
            @pl.when(idx + (DEPTH - 1) < NPAIR)
            def _():
                fire(tbl, idx + (DEPTH - 1), (k + DEPTH - 1) % DEPTH)
            drain(k)
            c0 = lo + 2 * idx
            c0f = jnp.minimum(c0, NCOLS - 2)
            s = lax.shift_right_logical(2 * idx, 5)

            def mscan(v, mcnt):
                w = pl.ds(pl.multiple_of(v * 16, 16), 16)
                cv = scol[s, w]
                m = (cv >= c0) & (cv <= c0 + 1)
                lane_full = slane[s, w] + ((cv - c0f) << 7)
                plsc.store_compressed(mst_lane.at[pl.ds(mcnt, 16)], lane_full, mask=m)
                plsc.store_compressed(mst_pos.at[pl.ds(mcnt, 16)], spos[s, w], mask=m)
                pc = plsc.all_reduce_population_count(m)[0]
                return jnp.minimum(mcnt + pc, 32)

            mcnt = lax.fori_loop(0, NSL, mscan, 0, unroll=False)

            lanes16 = mst_lane[pl.ds(0, 16)]
            poss16 = mst_pos[pl.ds(0, 16)]
            vmask0 = iota16 < mcnt
            slots0 = outcnt + iota16
            plsc.store_scatter(pos_st, [zeros16, slots0], poss16, mask=vmask0)
            for i in range(8):
                @pl.when(i < mcnt)
                def _(i=i):
                    lv = jnp.full((16,), lanes16[i], jnp.int32)
                    slot = outcnt + i
                    for g4 in range(EMBED // 16):
                        ev = iota16 + 16 * g4
                        vals = plsc.load_gather(bufs[k], [ev, lv])
                        rows_st[slot, pl.ds(16 * g4, 16)] = vals

            def epass(pp, carry):
                w = pl.ds(pl.multiple_of(pp * 16, 16), 16)
                lanes = mst_lane[w]
                poss = mst_pos[w]
                vmask = (iota16 + pp * 16) < mcnt
                slots = outcnt + pp * 16 + iota16
                plsc.store_scatter(pos_st, [zeros16, slots], poss, mask=vmask)
                for e in range(EMBED):
                    ev = jnp.full((16,), e, jnp.int32)
                    vals = plsc.load_gather(bufs[k], [ev, lanes], mask=vmask)
                    plsc.store_scatter(rows_st, [slots, ev], vals, mask=vmask)
                return carry

            ep_trips = jnp.where(mcnt > 8, (mcnt + 15) // 16, 0)
            lax.fori_loop(0, ep_trips, epass, 0, unroll=False)

            newcnt = outcnt + mcnt

            @pl.when(newcnt >= FLUSH_AT)
            def _():
                flush()

            return jnp.where(newcnt >= FLUSH_AT, 0, newcnt)

        def grp(g, outcnt):
            for k in range(DEPTH):
                outcnt = do_pair(DEPTH * g + k, k, outcnt)
            return outcnt

        lax.fori_loop(0, NPAIR // DEPTH, grp, 0, unroll=False)
        flush()

    for k in range(DEPTH - 1):
        fire(ut_hbm, k, k)
    pltpu.sync_copy(uid_hbm, idsall)
    build_lists()
    process_table(ut_hbm, ou_hbm)

    for k in range(DEPTH - 1):
        fire(it_hbm, k, k)
    pltpu.sync_copy(iid_hbm, idsall)
    build_lists()
    process_table(it_hbm, oi_hbm)


_sc_gather = functools.partial(
    pl.kernel,
    mesh=plsc.VectorSubcoreMesh(core_axis_name="c", subcore_axis_name="s"),
    out_type=[
        jax.ShapeDtypeStruct((OUT_ROWS, HIDDEN), jnp.float32),
        jax.ShapeDtypeStruct((OUT_ROWS, HIDDEN), jnp.float32),
    ],
    scratch_types=[
        pltpu.VMEM((BATCH,), jnp.int32),
        pltpu.VMEM((CAPF,), jnp.int32),
        pltpu.VMEM((CAPF,), jnp.int32),
        pltpu.VMEM((NSUB, SUBCAP), jnp.int32),
        pltpu.VMEM((NSUB, SUBCAP), jnp.int32),
        pltpu.VMEM((NSUB, SUBCAP), jnp.int32),
        pltpu.VMEM((EMBED, 2 * LANES), jnp.float32),
        pltpu.VMEM((EMBED, 2 * LANES), jnp.float32),
        pltpu.VMEM((EMBED, 2 * LANES), jnp.float32),
        pltpu.VMEM((EMBED, 2 * LANES), jnp.float32),
        pltpu.VMEM((STAGE, HIDDEN), jnp.float32),
        pltpu.VMEM((1, STAGE), jnp.int32),
        pltpu.VMEM((48,), jnp.int32),
        pltpu.VMEM((48,), jnp.int32),
        pltpu.SemaphoreType.DMA,
        pltpu.SemaphoreType.DMA,
        pltpu.SemaphoreType.DMA,
        pltpu.SemaphoreType.DMA,
    ],
    compiler_params=pltpu.CompilerParams(needs_layout_passes=False),
)(_sc_body)


def _mlp_body(u_ref, i_ref, w1u_ref, w1i_ref, b1_ref, w2_ref, b2_ref, o_ref):
    u = u_ref[:, :EMBED]
    i = i_ref[:, :EMBED]
    h = jnp.dot(u, w1u_ref[...], preferred_element_type=jnp.float32)
    h = h + jnp.dot(i, w1i_ref[...], preferred_element_type=jnp.float32)
    h = jnp.maximum(h + b1_ref[...], 0.0)
    p = jnp.sum(h * w2_ref[...], axis=1) + b2_ref[0, 0]
    o_ref[...] = jax.nn.sigmoid(p)


def kernel(user_ids, item_ids, user_table, item_table, W1, b1, W2, b2):
    uid = user_ids.astype(jnp.int32)
    iid = item_ids.astype(jnp.int32)
    ut_t = user_table.T
    it_t = item_table.T
    u_rows, i_rows = _sc_gather(uid, iid, ut_t, it_t)

    w1u = W1[:, :EMBED].T
    w1i = W1[:, EMBED:].T
    b1r = b1.reshape(1, HIDDEN)
    w2r = W2.reshape(1, HIDDEN)
    b2r = b2.reshape(1, 1)

    out = pl.pallas_call(
        _mlp_body,
        grid=(BATCH // BM,),
        in_specs=[
            pl.BlockSpec((BM, HIDDEN), lambda m: (m, 0)),
            pl.BlockSpec((BM, HIDDEN), lambda m: (m, 0)),
            pl.BlockSpec((EMBED, HIDDEN), lambda m: (0, 0)),
            pl.BlockSpec((EMBED, HIDDEN), lambda m: (0, 0)),
            pl.BlockSpec((1, HIDDEN), lambda m: (0, 0)),
            pl.BlockSpec((1, HIDDEN), lambda m: (0, 0)),
            pl.BlockSpec((1, 1), lambda m: (0, 0)),
        ],
        out_specs=pl.BlockSpec((BM,), lambda m: (m,)),
        out_shape=jax.ShapeDtypeStruct((BATCH,), jnp.float32),
    )(u_rows, i_rows, w1u, w1i, b1r, w2r, b2r)
    return out


# speedup vs baseline: 2.1881x; 2.1881x over previous
import functools

import jax
import jax.numpy as jnp
from jax import lax
from jax.experimental import pallas as pl
from jax.experimental.pallas import tpu as pltpu
from jax.experimental.pallas import tpu_sc as plsc

BATCH = 16384
EMBED = 64
HIDDEN = 128
LANES = 128
NC = 2
NS = 16
NW = NC * NS
B_PER_W = BATCH // NW
BM = 2048


DEPTH = 4


def _gather_table(tbl_t, ids_v, rows_v, cols, sems):
    NG = B_PER_W // 16

    def fire(idval, buf, sem):
        off = pl.multiple_of((idval >> 7) * LANES, LANES)
        pltpu.async_copy(tbl_t.at[:, pl.ds(off, LANES)], buf, sem)

    def drain(buf, sem):
        pltpu.make_async_copy(tbl_t.at[:, pl.ds(0, LANES)], buf, sem).wait()

    idv0 = ids_v[pl.ds(0, 16)]
    for k in range(DEPTH - 1):
        fire(idv0[k], cols[k], sems[k])

    def body(G, carry):
        g16 = pl.multiple_of(G * 16, 16)
        idv = ids_v[pl.ds(g16, 16)]
        nidv = ids_v[pl.ds(g16 + 16, 16)]
        for k in range(16):
            pk, nk = k % DEPTH, (k + DEPTH - 1) % DEPTH
            nxt = idv[k + DEPTH - 1] if k < 16 - (DEPTH - 1) else nidv[k - 17 + DEPTH]
            if k < 16 - (DEPTH - 1):
                fire(nxt, cols[nk], sems[nk])
            else:
                @pl.when(G + 1 < NG)
                def _():
                    fire(nxt, cols[nk], sems[nk])
            drain(cols[pk], sems[pk])
            lane = jnp.bitwise_and(idv[k], LANES - 1)
            lv = jnp.full((16,), lane, dtype=jnp.int32)
            for g in range(EMBED // 16):
                ev = lax.iota(jnp.int32, 16) + (16 * g)
                vals = plsc.load_gather(cols[pk], [ev, lv])
                rows_v[g16 + k, pl.ds(16 * g, 16)] = vals
        return carry

    lax.fori_loop(0, NG, body, 0, unroll=False)


def _sc_gather_body(uid_hbm, iid_hbm, ut_hbm, it_hbm, ou_hbm, oi_hbm,
                    uid_s, iid_s, rows_v, col0, col1, col2, col3,
                    sem0, sem1, sem2, sem3):
    wid = lax.axis_index("s") * NC + lax.axis_index("c")
    base = wid * B_PER_W
    pltpu.sync_copy(uid_hbm.at[pl.ds(base, B_PER_W)], uid_s.at[pl.ds(0, B_PER_W)])
    pltpu.sync_copy(iid_hbm.at[pl.ds(base, B_PER_W)], iid_s.at[pl.ds(0, B_PER_W)])

    cols = (col0, col1, col2, col3)
    sems = (sem0, sem1, sem2, sem3)
    _gather_table(ut_hbm, uid_s, rows_v, cols, sems)
    pltpu.sync_copy(rows_v, ou_hbm.at[pl.ds(base, B_PER_W)])

    _gather_table(it_hbm, iid_s, rows_v, cols, sems)
    pltpu.sync_copy(rows_v, oi_hbm.at[pl.ds(base, B_PER_W)])


_sc_gather = functools.partial(
    pl.kernel,
    mesh=plsc.VectorSubcoreMesh(core_axis_name="c", subcore_axis_name="s"),
    out_type=[
        jax.ShapeDtypeStruct((BATCH, EMBED), jnp.float32),
        jax.ShapeDtypeStruct((BATCH, EMBED), jnp.float32),
    ],
    scratch_types=[
        pltpu.VMEM((B_PER_W + 16,), jnp.int32),
        pltpu.VMEM((B_PER_W + 16,), jnp.int32),
        pltpu.VMEM((B_PER_W, EMBED), jnp.float32),
        pltpu.VMEM((EMBED, LANES), jnp.float32),
        pltpu.VMEM((EMBED, LANES), jnp.float32),
        pltpu.VMEM((EMBED, LANES), jnp.float32),
        pltpu.VMEM((EMBED, LANES), jnp.float32),
        pltpu.SemaphoreType.DMA,
        pltpu.SemaphoreType.DMA,
        pltpu.SemaphoreType.DMA,
        pltpu.SemaphoreType.DMA,
    ],
    compiler_params=pltpu.CompilerParams(needs_layout_passes=False),
)(_sc_gather_body)


def _mlp_body(u_ref, i_ref, w1u_ref, w1i_ref, b1_ref, w2_ref, b2_ref, o_ref):
    h = jnp.dot(u_ref[...], w1u_ref[...], preferred_element_type=jnp.float32)
    h = h + jnp.dot(i_ref[...], w1i_ref[...], preferred_element_type=jnp.float32)
    h = jnp.maximum(h + b1_ref[...], 0.0)
    p = jnp.sum(h * w2_ref[...], axis=1) + b2_ref[0, 0]
    o_ref[...] = jax.nn.sigmoid(p)


def kernel(user_ids, item_ids, user_table, item_table, W1, b1, W2, b2):
    uid = user_ids.astype(jnp.int32)
    iid = item_ids.astype(jnp.int32)
    ut_t = user_table.T
    it_t = item_table.T
    u_rows, i_rows = _sc_gather(uid, iid, ut_t, it_t)

    w1u = W1[:, :EMBED].T
    w1i = W1[:, EMBED:].T
    b1r = b1.reshape(1, HIDDEN)
    w2r = W2.reshape(1, HIDDEN)
    b2r = b2.reshape(1, 1)

    out = pl.pallas_call(
        _mlp_body,
        grid=(BATCH // BM,),
        in_specs=[
            pl.BlockSpec((BM, EMBED), lambda m: (m, 0)),
            pl.BlockSpec((BM, EMBED), lambda m: (m, 0)),
            pl.BlockSpec((EMBED, HIDDEN), lambda m: (0, 0)),
            pl.BlockSpec((EMBED, HIDDEN), lambda m: (0, 0)),
            pl.BlockSpec((1, HIDDEN), lambda m: (0, 0)),
            pl.BlockSpec((1, HIDDEN), lambda m: (0, 0)),
            pl.BlockSpec((1, 1), lambda m: (0, 0)),
        ],
        out_specs=pl.BlockSpec((BM,), lambda m: (m,)),
        out_shape=jax.ShapeDtypeStruct((BATCH,), jnp.float32),
    )(u_rows, i_rows, w1u, w1i, b1r, w2r, b2r)
    return out
